# Initial kernel scaffold; baseline (speedup 1.0000x reference)
import jax
import jax.numpy as jnp
from jax.experimental import pallas as pl


def kernel(x, edge_index, batch, W1, b1, W2, b2):
    raise NotImplementedError("write your pallas kernel here")


# speedup vs baseline: 3.5619x; 3.5619x over previous
import functools

import jax
import jax.numpy as jnp
from jax import lax
from jax.experimental import pallas as pl
from jax.experimental.pallas import tpu as pltpu
from jax.experimental.pallas import tpu_sc as plsc

N = 10000
NP = 10240
E = 320000
D = 128
L = 4
G = 64

NC = 2
NS = 16
NW = NC * NS

EPT = E // NW
K = 40
NCHUNK = EPT // K
NPT = NP // NS


def _agg_body(h_hbm, src_hbm, dst_hbm, zeros_hbm, out_hbm,
              sidx, dchunk, rows, acc, gsem):
    c = lax.axis_index("c")
    s = lax.axis_index("s")
    wid = c * NS + s
    ebase = wid * EPT
    nbase = s * NPT

    pltpu.sync_copy(zeros_hbm.at[pl.ds(nbase, NPT)], acc.at[pl.ds(nbase, NPT)])
    pltpu.sync_copy(src_hbm.at[pl.ds(ebase, EPT)], sidx)
    plsc.subcore_barrier()

    @pl.loop(0, NCHUNK)
    def _chunk(j):
        off = j * K
        pltpu.sync_copy(dst_hbm.at[pl.ds(ebase + off, K)], dchunk)
        pltpu.async_copy(h_hbm.at[sidx.at[pl.ds(off, K)]], rows, gsem).wait()
        pltpu.sync_copy(rows, acc.at[dchunk], add=True)

    plsc.subcore_barrier()
    pltpu.sync_copy(acc.at[pl.ds(nbase, NPT)], out_hbm.at[c, pl.ds(nbase, NPT)])


_agg = pl.kernel(
    _agg_body,
    out_type=jax.ShapeDtypeStruct((NC, NP, D), jnp.float32),
    mesh=plsc.VectorSubcoreMesh(
        core_axis_name="c", subcore_axis_name="s",
        num_cores=NC, num_subcores=NS),
    scratch_types=[
        pltpu.VMEM((EPT,), jnp.int32),
        pltpu.VMEM((K,), jnp.int32),
        pltpu.VMEM((K, D), jnp.float32),
        pltpu.VMEM_SHARED((NP, D), jnp.float32),
        pltpu.SemaphoreType.DMA,
    ],
)


def _mlp_body(last, h_ref, p0_ref, p1_ref, w1_ref, b1_ref, w2_ref, b2_ref,
              o_ref):
    hin = h_ref[...] + p0_ref[...] + p1_ref[...]
    t = jnp.dot(hin, w1_ref[...], preferred_element_type=jnp.float32)
    t = jnp.maximum(t + b1_ref[...], 0.0)
    o = jnp.dot(t, w2_ref[...], preferred_element_type=jnp.float32)
    o = o + b2_ref[...]
    if not last:
        o = jnp.maximum(o, 0.0)
    o_ref[...] = o


_RB = 1024
_NB = NP // _RB


def _mlp(h, p0, p1, w1, b1, w2, b2, last):
    row_spec = pl.BlockSpec((_RB, D), lambda i: (i, 0))
    full = pl.BlockSpec((D, D), lambda i: (0, 0))
    bias = pl.BlockSpec((1, D), lambda i: (0, 0))
    return pl.pallas_call(
        functools.partial(_mlp_body, last),
        grid=(_NB,),
        in_specs=[row_spec, row_spec, row_spec, full, bias, full, bias],
        out_specs=row_spec,
        out_shape=jax.ShapeDtypeStruct((NP, D), jnp.float32),
    )(h, p0, p1, w1, b1.reshape(1, D), w2, b2.reshape(1, D))


def _pool_body(h_ref, b_ref, o_ref, sums, cnts):
    i = pl.program_id(0)

    @pl.when(i == 0)
    def _():
        sums[...] = jnp.zeros_like(sums)
        cnts[...] = jnp.zeros_like(cnts)

    bvec = b_ref[0]
    gids = lax.broadcasted_iota(jnp.int32, (G, _RB), 0)
    m = (bvec == gids).astype(jnp.float32)
    h = h_ref[...]
    sums[...] += lax.dot_general(m, h, (((1,), (0,)), ((), ())),
                                 preferred_element_type=jnp.float32)
    cnts[...] += lax.dot_general(m, jnp.ones_like(h), (((1,), (0,)), ((), ())),
                                 preferred_element_type=jnp.float32)

    @pl.when(i == _NB - 1)
    def _():
        o_ref[...] = sums[...] / jnp.maximum(cnts[...], 1.0)


def _pool(h, batch3d):
    return pl.pallas_call(
        _pool_body,
        grid=(_NB,),
        in_specs=[
            pl.BlockSpec((_RB, D), lambda i: (i, 0)),
            pl.BlockSpec((1, 1, _RB), lambda i: (i, 0, 0)),
        ],
        out_specs=pl.BlockSpec((G, D), lambda i: (0, 0)),
        out_shape=jax.ShapeDtypeStruct((G, D), jnp.float32),
        scratch_shapes=[
            pltpu.VMEM((G, D), jnp.float32),
            pltpu.VMEM((G, D), jnp.float32),
        ],
    )(h, batch3d)


def kernel(x, edge_index, batch, W1, b1, W2, b2):
    src = edge_index[0]
    dst = edge_index[1]
    zeros = jnp.zeros((NP, D), jnp.float32)
    h = jnp.pad(x, ((0, NP - N), (0, 0)))
    batch3d = jnp.pad(batch, (0, NP - N), constant_values=G).reshape(
        _NB, 1, _RB)
    for l in range(L):
        parts = _agg(h, src, dst, zeros)
        h = _mlp(h, parts[0], parts[1], W1[l], b1[l], W2[l], b2[l],
                 last=(l == L - 1))
    return _pool(h, batch3d)


# speedup vs baseline: 7.2611x; 2.0385x over previous
import functools

import jax
import jax.numpy as jnp
from jax import lax
from jax.experimental import pallas as pl
from jax.experimental.pallas import tpu as pltpu
from jax.experimental.pallas import tpu_sc as plsc

N = 10000
NP = 10240
E = 320000
D = 128
L = 4
G = 64

NC = 2
NS = 16
NW = NC * NS

EPT = E // NW
K = 200
NCHUNK = EPT // K
NPT = NP // NS


def _agg_body(h_hbm, src_hbm, dst_hbm, zeros_hbm, out_hbm,
              sidx, dchunk, rows, acc, gsem):
    c = lax.axis_index("c")
    s = lax.axis_index("s")
    wid = c * NS + s
    ebase = wid * EPT
    nbase = s * NPT

    pltpu.sync_copy(zeros_hbm.at[pl.ds(nbase, NPT)], acc.at[pl.ds(nbase, NPT)])
    pltpu.sync_copy(src_hbm.at[pl.ds(ebase, EPT)], sidx)
    plsc.subcore_barrier()

    @pl.loop(0, NCHUNK)
    def _chunk(j):
        off = j * K
        pltpu.sync_copy(dst_hbm.at[pl.ds(ebase + off, K)], dchunk)
        pltpu.async_copy(h_hbm.at[sidx.at[pl.ds(off, K)]], rows, gsem).wait()
        pltpu.sync_copy(rows, acc.at[dchunk], add=True)

    plsc.subcore_barrier()
    pltpu.sync_copy(acc.at[pl.ds(nbase, NPT)], out_hbm.at[c, pl.ds(nbase, NPT)])


_agg = pl.kernel(
    _agg_body,
    out_type=jax.ShapeDtypeStruct((NC, NP, D), jnp.float32),
    mesh=plsc.VectorSubcoreMesh(
        core_axis_name="c", subcore_axis_name="s",
        num_cores=NC, num_subcores=NS),
    scratch_types=[
        pltpu.VMEM((EPT,), jnp.int32),
        pltpu.VMEM((K,), jnp.int32),
        pltpu.VMEM((K, D), jnp.float32),
        pltpu.VMEM_SHARED((NP, D), jnp.float32),
        pltpu.SemaphoreType.DMA,
    ],
)


def _mlp_body(last, h_ref, p0_ref, p1_ref, w1_ref, b1_ref, w2_ref, b2_ref,
              o_ref):
    hin = h_ref[...] + p0_ref[...] + p1_ref[...]
    t = jnp.dot(hin, w1_ref[...], preferred_element_type=jnp.float32)
    t = jnp.maximum(t + b1_ref[...], 0.0)
    o = jnp.dot(t, w2_ref[...], preferred_element_type=jnp.float32)
    o = o + b2_ref[...]
    if not last:
        o = jnp.maximum(o, 0.0)
    o_ref[...] = o


_RB = 1024
_NB = NP // _RB


def _mlp(h, p0, p1, w1, b1, w2, b2, last):
    row_spec = pl.BlockSpec((_RB, D), lambda i: (i, 0))
    full = pl.BlockSpec((D, D), lambda i: (0, 0))
    bias = pl.BlockSpec((1, D), lambda i: (0, 0))
    return pl.pallas_call(
        functools.partial(_mlp_body, last),
        grid=(_NB,),
        in_specs=[row_spec, row_spec, row_spec, full, bias, full, bias],
        out_specs=row_spec,
        out_shape=jax.ShapeDtypeStruct((NP, D), jnp.float32),
    )(h, p0, p1, w1, b1.reshape(1, D), w2, b2.reshape(1, D))


def _pool_body(h_ref, b_ref, o_ref, sums, cnts):
    i = pl.program_id(0)

    @pl.when(i == 0)
    def _():
        sums[...] = jnp.zeros_like(sums)
        cnts[...] = jnp.zeros_like(cnts)

    bvec = b_ref[0]
    gids = lax.broadcasted_iota(jnp.int32, (G, _RB), 0)
    m = (bvec == gids).astype(jnp.float32)
    h = h_ref[...]
    sums[...] += lax.dot_general(m, h, (((1,), (0,)), ((), ())),
                                 preferred_element_type=jnp.float32)
    cnts[...] += lax.dot_general(m, jnp.ones_like(h), (((1,), (0,)), ((), ())),
                                 preferred_element_type=jnp.float32)

    @pl.when(i == _NB - 1)
    def _():
        o_ref[...] = sums[...] / jnp.maximum(cnts[...], 1.0)


def _pool(h, batch3d):
    return pl.pallas_call(
        _pool_body,
        grid=(_NB,),
        in_specs=[
            pl.BlockSpec((_RB, D), lambda i: (i, 0)),
            pl.BlockSpec((1, 1, _RB), lambda i: (i, 0, 0)),
        ],
        out_specs=pl.BlockSpec((G, D), lambda i: (0, 0)),
        out_shape=jax.ShapeDtypeStruct((G, D), jnp.float32),
        scratch_shapes=[
            pltpu.VMEM((G, D), jnp.float32),
            pltpu.VMEM((G, D), jnp.float32),
        ],
    )(h, batch3d)


def kernel(x, edge_index, batch, W1, b1, W2, b2):
    src = edge_index[0]
    dst = edge_index[1]
    zeros = jnp.zeros((NP, D), jnp.float32)
    h = jnp.pad(x, ((0, NP - N), (0, 0)))
    batch3d = jnp.pad(batch, (0, NP - N), constant_values=G).reshape(
        _NB, 1, _RB)
    for l in range(L):
        parts = _agg(h, src, dst, zeros)
        h = _mlp(h, parts[0], parts[1], W1[l], b1[l], W2[l], b2[l],
                 last=(l == L - 1))
    return _pool(h, batch3d)


# speedup vs baseline: 9.9840x; 1.3750x over previous
import functools

import jax
import jax.numpy as jnp
from jax import lax
from jax.experimental import pallas as pl
from jax.experimental.pallas import tpu as pltpu
from jax.experimental.pallas import tpu_sc as plsc

N = 10000
NP = 10240
E = 320000
D = 128
L = 4
G = 64

NC = 2
NS = 16
NW = NC * NS

EPT = E // NW
K = 80
NCHUNK = EPT // K
NPT = NP // NS


def _agg_body(h_hbm, src_hbm, dst_hbm, zeros_hbm, out_hbm,
              sidx, didx, rows0, rows1, acc, gsem0, gsem1, ssem0, ssem1):
    c = lax.axis_index("c")
    s = lax.axis_index("s")
    wid = c * NS + s
    ebase = wid * EPT
    nbase = s * NPT

    rows = (rows0, rows1)
    gsem = (gsem0, gsem1)
    ssem = (ssem0, ssem1)

    pltpu.sync_copy(zeros_hbm.at[pl.ds(nbase, NPT)], acc.at[pl.ds(nbase, NPT)])
    pltpu.sync_copy(src_hbm.at[pl.ds(ebase, EPT)], sidx)
    pltpu.sync_copy(dst_hbm.at[wid], didx)
    plsc.subcore_barrier()

    def _drain(buf, sem):
        pltpu.make_async_copy(h_hbm.at[pl.ds(0, K)], buf, sem).wait()

    pltpu.async_copy(h_hbm.at[sidx.at[pl.ds(0, K)]], rows0, gsem0)

    def _stage(j, p, pn, last):
        if not last:
            @pl.when(j >= 1)
            def _():
                _drain(rows[pn], ssem[pn])
            pltpu.async_copy(
                h_hbm.at[sidx.at[pl.ds((j + 1) * K, K)]], rows[pn], gsem[pn])
        _drain(rows[p], gsem[p])
        pltpu.async_copy(rows[p], acc.at[didx.at[j]], ssem[p], add=True)

    @pl.loop(0, NCHUNK - 1, step=2)
    def _chunk(j):
        _stage(j, 0, 1, False)
        _stage(j + 1, 1, 0, False)

    _stage(NCHUNK - 1, (NCHUNK - 1) % 2, NCHUNK % 2, True)
    _drain(rows[1], ssem[1])
    _drain(rows[0], ssem[0])

    plsc.subcore_barrier()
    pltpu.sync_copy(acc.at[pl.ds(nbase, NPT)], out_hbm.at[c, pl.ds(nbase, NPT)])


_agg = pl.kernel(
    _agg_body,
    out_type=jax.ShapeDtypeStruct((NC, NP, D), jnp.float32),
    mesh=plsc.VectorSubcoreMesh(
        core_axis_name="c", subcore_axis_name="s",
        num_cores=NC, num_subcores=NS),
    scratch_types=[
        pltpu.VMEM((EPT,), jnp.int32),
        pltpu.VMEM((NCHUNK, K), jnp.int32),
        pltpu.VMEM((K, D), jnp.float32),
        pltpu.VMEM((K, D), jnp.float32),
        pltpu.VMEM_SHARED((NP, D), jnp.float32),
        pltpu.SemaphoreType.DMA,
        pltpu.SemaphoreType.DMA,
        pltpu.SemaphoreType.DMA,
        pltpu.SemaphoreType.DMA,
    ],
)


def _mlp_body(last, h_ref, p0_ref, p1_ref, w1_ref, b1_ref, w2_ref, b2_ref,
              o_ref):
    hin = h_ref[...] + p0_ref[...] + p1_ref[...]
    t = jnp.dot(hin, w1_ref[...], preferred_element_type=jnp.float32)
    t = jnp.maximum(t + b1_ref[...], 0.0)
    o = jnp.dot(t, w2_ref[...], preferred_element_type=jnp.float32)
    o = o + b2_ref[...]
    if not last:
        o = jnp.maximum(o, 0.0)
    o_ref[...] = o


_RB = 1024
_NB = NP // _RB


def _mlp(h, p0, p1, w1, b1, w2, b2, last):
    row_spec = pl.BlockSpec((_RB, D), lambda i: (i, 0))
    full = pl.BlockSpec((D, D), lambda i: (0, 0))
    bias = pl.BlockSpec((1, D), lambda i: (0, 0))
    return pl.pallas_call(
        functools.partial(_mlp_body, last),
        grid=(_NB,),
        in_specs=[row_spec, row_spec, row_spec, full, bias, full, bias],
        out_specs=row_spec,
        out_shape=jax.ShapeDtypeStruct((NP, D), jnp.float32),
    )(h, p0, p1, w1, b1.reshape(1, D), w2, b2.reshape(1, D))


def _pool_body(h_ref, b_ref, o_ref, sums, cnts):
    i = pl.program_id(0)

    @pl.when(i == 0)
    def _():
        sums[...] = jnp.zeros_like(sums)
        cnts[...] = jnp.zeros_like(cnts)

    bvec = b_ref[0]
    gids = lax.broadcasted_iota(jnp.int32, (G, _RB), 0)
    m = (bvec == gids).astype(jnp.float32)
    h = h_ref[...]
    sums[...] += lax.dot_general(m, h, (((1,), (0,)), ((), ())),
                                 preferred_element_type=jnp.float32)
    cnts[...] += lax.dot_general(m, jnp.ones_like(h), (((1,), (0,)), ((), ())),
                                 preferred_element_type=jnp.float32)

    @pl.when(i == _NB - 1)
    def _():
        o_ref[...] = sums[...] / jnp.maximum(cnts[...], 1.0)


def _pool(h, batch3d):
    return pl.pallas_call(
        _pool_body,
        grid=(_NB,),
        in_specs=[
            pl.BlockSpec((_RB, D), lambda i: (i, 0)),
            pl.BlockSpec((1, 1, _RB), lambda i: (i, 0, 0)),
        ],
        out_specs=pl.BlockSpec((G, D), lambda i: (0, 0)),
        out_shape=jax.ShapeDtypeStruct((G, D), jnp.float32),
        scratch_shapes=[
            pltpu.VMEM((G, D), jnp.float32),
            pltpu.VMEM((G, D), jnp.float32),
        ],
    )(h, batch3d)


def kernel(x, edge_index, batch, W1, b1, W2, b2):
    src = edge_index[0]
    dst = edge_index[1].reshape(NW, NCHUNK, K)
    zeros = jnp.zeros((NP, D), jnp.float32)
    h = jnp.pad(x, ((0, NP - N), (0, 0)))
    batch3d = jnp.pad(batch, (0, NP - N), constant_values=G).reshape(
        _NB, 1, _RB)
    for l in range(L):
        parts = _agg(h, src, dst, zeros)
        h = _mlp(h, parts[0], parts[1], W1[l], b1[l], W2[l], b2[l],
                 last=(l == L - 1))
    return _pool(h, batch3d)


# speedup vs baseline: 11.5585x; 1.1577x over previous
import functools

import jax
import jax.numpy as jnp
from jax import lax
from jax.experimental import pallas as pl
from jax.experimental.pallas import tpu as pltpu
from jax.experimental.pallas import tpu_sc as plsc

N = 10000
NP = 10240
E = 320000
D = 128
L = 4
G = 64

NC = 2
NS = 16
NW = NC * NS

EPT = E // NW
K = 80
NCHUNK = EPT // K
NPT = NP // NS


R = 3
RI = 6
P = 4


def _agg_body(h_hbm, src_hbm, dst_hbm, zeros_hbm, out_hbm, *scratch):
    sidx = scratch[0]
    ibuf = scratch[1]
    rows = scratch[2:2 + R]
    acc = scratch[2 + R]
    gsem = scratch[3 + R:3 + 2 * R]
    ssem = scratch[3 + 2 * R:3 + 3 * R]
    isem = scratch[3 + 3 * R:3 + 3 * R + RI]

    c = lax.axis_index("c")
    s = lax.axis_index("s")
    wid = c * NS + s
    ebase = wid * EPT
    nbase = s * NPT

    def _idx_fetch(m, slot):
        pltpu.async_copy(dst_hbm.at[pl.ds(ebase + m * K, K)],
                         ibuf.at[slot], isem[slot])

    pltpu.sync_copy(zeros_hbm.at[pl.ds(nbase, NPT)], acc.at[pl.ds(nbase, NPT)])
    pltpu.sync_copy(src_hbm.at[pl.ds(ebase, EPT)], sidx)
    for m in range(P):
        _idx_fetch(m, m % RI)
    plsc.subcore_barrier()

    def _drain(buf, sem):
        pltpu.make_async_copy(h_hbm.at[pl.ds(0, K)], buf, sem).wait()

    def _draini(q):
        pltpu.make_async_copy(dst_hbm.at[pl.ds(0, K)], ibuf.at[q],
                              isem[q]).wait()

    pltpu.async_copy(h_hbm.at[sidx.at[pl.ds(0, K)]], rows[0], gsem[0])

    def _stage(j, b, bn, q, fetch, last):
        if not last:
            @pl.when(j + 1 >= R)
            def _():
                _drain(rows[bn], ssem[bn])
        if fetch:
            _idx_fetch(j + P, (q + P) % RI)
        if not last:
            pltpu.async_copy(
                h_hbm.at[sidx.at[pl.ds((j + 1) * K, K)]], rows[bn], gsem[bn])
        _drain(rows[b], gsem[b])
        _draini(q)
        pltpu.async_copy(rows[b], acc.at[ibuf.at[q]], ssem[b], add=True)

    MAIN = (NCHUNK - 1) // RI * RI

    @pl.loop(0, MAIN, step=RI)
    def _chunk(j):
        for r in range(RI):
            _stage(j + r, r % R, (r + 1) % R, r, True, False)

    for j in range(MAIN, NCHUNK):
        _stage(j, j % R, (j + 1) % R, j % RI, j + P < NCHUNK,
               j == NCHUNK - 1)
    for b in range(R):
        _drain(rows[b], ssem[b])

    plsc.subcore_barrier()
    pltpu.sync_copy(acc.at[pl.ds(nbase, NPT)], out_hbm.at[c, pl.ds(nbase, NPT)])


_agg = pl.kernel(
    _agg_body,
    out_type=jax.ShapeDtypeStruct((NC, NP, D), jnp.float32),
    mesh=plsc.VectorSubcoreMesh(
        core_axis_name="c", subcore_axis_name="s",
        num_cores=NC, num_subcores=NS),
    scratch_types=(
        [pltpu.VMEM((EPT,), jnp.int32),
         pltpu.VMEM((RI, K), jnp.int32)]
        + [pltpu.VMEM((K, D), jnp.float32) for _ in range(R)]
        + [pltpu.VMEM_SHARED((NP, D), jnp.float32)]
        + [pltpu.SemaphoreType.DMA for _ in range(2 * R)]
        + [pltpu.SemaphoreType.DMA for _ in range(RI)]
    ),
)


def _mlp_body(last, h_ref, p0_ref, p1_ref, w1_ref, b1_ref, w2_ref, b2_ref,
              o_ref):
    hin = h_ref[...] + p0_ref[...] + p1_ref[...]
    t = jnp.dot(hin, w1_ref[...], preferred_element_type=jnp.float32)
    t = jnp.maximum(t + b1_ref[...], 0.0)
    o = jnp.dot(t, w2_ref[...], preferred_element_type=jnp.float32)
    o = o + b2_ref[...]
    if not last:
        o = jnp.maximum(o, 0.0)
    o_ref[...] = o


_RB = 1024
_NB = NP // _RB


def _mlp(h, p0, p1, w1, b1, w2, b2, last):
    row_spec = pl.BlockSpec((_RB, D), lambda i: (i, 0))
    full = pl.BlockSpec((D, D), lambda i: (0, 0))
    bias = pl.BlockSpec((1, D), lambda i: (0, 0))
    return pl.pallas_call(
        functools.partial(_mlp_body, last),
        grid=(_NB,),
        in_specs=[row_spec, row_spec, row_spec, full, bias, full, bias],
        out_specs=row_spec,
        out_shape=jax.ShapeDtypeStruct((NP, D), jnp.float32),
    )(h, p0, p1, w1, b1.reshape(1, D), w2, b2.reshape(1, D))


def _pool_body(h_ref, b_ref, o_ref, sums, cnts):
    i = pl.program_id(0)

    @pl.when(i == 0)
    def _():
        sums[...] = jnp.zeros_like(sums)
        cnts[...] = jnp.zeros_like(cnts)

    bvec = b_ref[0]
    gids = lax.broadcasted_iota(jnp.int32, (G, _RB), 0)
    m = (bvec == gids).astype(jnp.float32)
    h = h_ref[...]
    sums[...] += lax.dot_general(m, h, (((1,), (0,)), ((), ())),
                                 preferred_element_type=jnp.float32)
    cnts[...] += lax.dot_general(m, jnp.ones_like(h), (((1,), (0,)), ((), ())),
                                 preferred_element_type=jnp.float32)

    @pl.when(i == _NB - 1)
    def _():
        o_ref[...] = sums[...] / jnp.maximum(cnts[...], 1.0)


def _pool(h, batch3d):
    return pl.pallas_call(
        _pool_body,
        grid=(_NB,),
        in_specs=[
            pl.BlockSpec((_RB, D), lambda i: (i, 0)),
            pl.BlockSpec((1, 1, _RB), lambda i: (i, 0, 0)),
        ],
        out_specs=pl.BlockSpec((G, D), lambda i: (0, 0)),
        out_shape=jax.ShapeDtypeStruct((G, D), jnp.float32),
        scratch_shapes=[
            pltpu.VMEM((G, D), jnp.float32),
            pltpu.VMEM((G, D), jnp.float32),
        ],
    )(h, batch3d)


def kernel(x, edge_index, batch, W1, b1, W2, b2):
    src = edge_index[0]
    dst = edge_index[1]
    zeros = jnp.zeros((NP, D), jnp.float32)
    h = jnp.pad(x, ((0, NP - N), (0, 0)))
    batch3d = jnp.pad(batch, (0, NP - N), constant_values=G).reshape(
        _NB, 1, _RB)
    for l in range(L):
        parts = _agg(h, src, dst, zeros)
        h = _mlp(h, parts[0], parts[1], W1[l], b1[l], W2[l], b2[l],
                 last=(l == L - 1))
    return _pool(h, batch3d)


# speedup vs baseline: 11.6948x; 1.0118x over previous
import jax
import jax.numpy as jnp
from jax import lax
from jax.experimental import pallas as pl
from jax.experimental.pallas import tpu as pltpu
from jax.experimental.pallas import tpu_sc as plsc

N = 10000
NP = 10240
E = 320000
D = 128
L = 4
G = 64

NC = 2
NS = 16
NW = NC * NS

EPT = E // NW
K = 80
NCHUNK = EPT // K
NPT = NP // NS


R = 3
RI = 6
P = 4


def _agg_body(h_hbm, src_hbm, dst_hbm, zeros_hbm, out_hbm, *scratch):
    sidx = scratch[0]
    ibuf = scratch[1]
    rows = scratch[2:2 + R]
    acc = scratch[2 + R]
    gsem = scratch[3 + R:3 + 2 * R]
    ssem = scratch[3 + 2 * R:3 + 3 * R]
    isem = scratch[3 + 3 * R:3 + 3 * R + RI]

    c = lax.axis_index("c")
    s = lax.axis_index("s")
    wid = c * NS + s
    ebase = wid * EPT
    nbase = s * NPT

    def _idx_fetch(m, slot):
        pltpu.async_copy(dst_hbm.at[pl.ds(ebase + m * K, K)],
                         ibuf.at[slot], isem[slot])

    pltpu.sync_copy(zeros_hbm.at[pl.ds(nbase, NPT)], acc.at[pl.ds(nbase, NPT)])
    pltpu.sync_copy(src_hbm.at[pl.ds(ebase, EPT)], sidx)
    for m in range(P):
        _idx_fetch(m, m % RI)
    plsc.subcore_barrier()

    def _drain(buf, sem):
        pltpu.make_async_copy(h_hbm.at[pl.ds(0, K)], buf, sem).wait()

    def _draini(q):
        pltpu.make_async_copy(dst_hbm.at[pl.ds(0, K)], ibuf.at[q],
                              isem[q]).wait()

    pltpu.async_copy(h_hbm.at[sidx.at[pl.ds(0, K)]], rows[0], gsem[0])

    def _stage(j, b, bn, q, fetch, last):
        if not last:
            @pl.when(j + 1 >= R)
            def _():
                _drain(rows[bn], ssem[bn])
        if fetch:
            _idx_fetch(j + P, (q + P) % RI)
        if not last:
            pltpu.async_copy(
                h_hbm.at[sidx.at[pl.ds((j + 1) * K, K)]], rows[bn], gsem[bn])
        _drain(rows[b], gsem[b])
        _draini(q)
        pltpu.async_copy(rows[b], acc.at[ibuf.at[q]], ssem[b], add=True)

    MAIN = (NCHUNK - 1) // RI * RI

    @pl.loop(0, MAIN, step=RI)
    def _chunk(j):
        for r in range(RI):
            _stage(j + r, r % R, (r + 1) % R, r, True, False)

    for j in range(MAIN, NCHUNK):
        _stage(j, j % R, (j + 1) % R, j % RI, j + P < NCHUNK,
               j == NCHUNK - 1)
    for b in range(R):
        _drain(rows[b], ssem[b])

    plsc.subcore_barrier()
    pltpu.sync_copy(acc.at[pl.ds(nbase, NPT)], out_hbm.at[c, pl.ds(nbase, NPT)])


_agg = pl.kernel(
    _agg_body,
    out_type=jax.ShapeDtypeStruct((NC, NP, D), jnp.float32),
    mesh=plsc.VectorSubcoreMesh(
        core_axis_name="c", subcore_axis_name="s",
        num_cores=NC, num_subcores=NS),
    scratch_types=(
        [pltpu.VMEM((EPT,), jnp.int32),
         pltpu.VMEM((RI, K), jnp.int32)]
        + [pltpu.VMEM((K, D), jnp.float32) for _ in range(R)]
        + [pltpu.VMEM_SHARED((NP, D), jnp.float32)]
        + [pltpu.SemaphoreType.DMA for _ in range(2 * R)]
        + [pltpu.SemaphoreType.DMA for _ in range(RI)]
    ),
)


_RB = 1024
_NB = NP // _RB

_row = pl.BlockSpec((_RB, D), lambda i: (i, 0))
_full = pl.BlockSpec((D, D), lambda i: (0, 0))
_bias = pl.BlockSpec((1, D), lambda i: (0, 0))


def _premm_body(h_ref, w1_ref, b1_ref, o_ref):
    o_ref[...] = jnp.dot(h_ref[...], w1_ref[...],
                         preferred_element_type=jnp.float32) + b1_ref[...]


def _premm(h, w1, b1):
    return pl.pallas_call(
        _premm_body,
        grid=(_NB,),
        in_specs=[_row, _full, _bias],
        out_specs=_row,
        out_shape=jax.ShapeDtypeStruct((NP, D), jnp.float32),
    )(h, w1, b1.reshape(1, D))


def _finish(hw1, p0, p1, w1, w2, b2):
    t = hw1 + jnp.dot(p0 + p1, w1, preferred_element_type=jnp.float32)
    t = jnp.maximum(t, 0.0)
    return jnp.dot(t, w2, preferred_element_type=jnp.float32) + b2


def _mlp_body(hw1_ref, p0_ref, p1_ref, w1_ref, w2_ref, b2_ref, o_ref):
    o = _finish(hw1_ref[...], p0_ref[...], p1_ref[...], w1_ref[...],
                w2_ref[...], b2_ref[...])
    o_ref[...] = jnp.maximum(o, 0.0)


def _mlp(hw1, p0, p1, w1, w2, b2):
    return pl.pallas_call(
        _mlp_body,
        grid=(_NB,),
        in_specs=[_row, _row, _row, _full, _full, _bias],
        out_specs=_row,
        out_shape=jax.ShapeDtypeStruct((NP, D), jnp.float32),
    )(hw1, p0, p1, w1, w2, b2.reshape(1, D))


def _mlp_pool_body(hw1_ref, p0_ref, p1_ref, w1_ref, w2_ref, b2_ref, b_ref,
                   o_ref, sums, cnts):
    i = pl.program_id(0)

    @pl.when(i == 0)
    def _():
        sums[...] = jnp.zeros_like(sums)
        cnts[...] = jnp.zeros_like(cnts)

    o = _finish(hw1_ref[...], p0_ref[...], p1_ref[...], w1_ref[...],
                w2_ref[...], b2_ref[...])
    bvec = b_ref[0]
    gids = lax.broadcasted_iota(jnp.int32, (G, _RB), 0)
    m = (bvec == gids).astype(jnp.float32)
    sums[...] += lax.dot_general(m, o, (((1,), (0,)), ((), ())),
                                 preferred_element_type=jnp.float32)
    cnts[...] += lax.dot_general(m, jnp.ones_like(o), (((1,), (0,)), ((), ())),
                                 preferred_element_type=jnp.float32)

    @pl.when(i == _NB - 1)
    def _():
        o_ref[...] = sums[...] / jnp.maximum(cnts[...], 1.0)


def _mlp_pool(hw1, p0, p1, w1, w2, b2, batch3d):
    return pl.pallas_call(
        _mlp_pool_body,
        grid=(_NB,),
        in_specs=[_row, _row, _row, _full, _full, _bias,
                  pl.BlockSpec((1, 1, _RB), lambda i: (i, 0, 0))],
        out_specs=pl.BlockSpec((G, D), lambda i: (0, 0)),
        out_shape=jax.ShapeDtypeStruct((G, D), jnp.float32),
        scratch_shapes=[
            pltpu.VMEM((G, D), jnp.float32),
            pltpu.VMEM((G, D), jnp.float32),
        ],
    )(hw1, p0, p1, w1, w2, b2.reshape(1, D), batch3d)


def kernel(x, edge_index, batch, W1, b1, W2, b2):
    src = edge_index[0]
    dst = edge_index[1]
    zeros = jnp.zeros((NP, D), jnp.float32)
    h = jnp.pad(x, ((0, NP - N), (0, 0)))
    batch3d = jnp.pad(batch, (0, NP - N), constant_values=G).reshape(
        _NB, 1, _RB)
    for l in range(L):
        hw1 = _premm(h, W1[l], b1[l])
        parts = _agg(h, src, dst, zeros)
        if l < L - 1:
            h = _mlp(hw1, parts[0], parts[1], W1[l], W2[l], b2[l])
        else:
            return _mlp_pool(hw1, parts[0], parts[1], W1[l], W2[l], b2[l],
                             batch3d)


# speedup vs baseline: 12.0230x; 1.0281x over previous
import jax
import jax.numpy as jnp
from jax import lax
from jax.experimental import pallas as pl
from jax.experimental.pallas import tpu as pltpu
from jax.experimental.pallas import tpu_sc as plsc

N = 10000
NP = 10240
E = 320000
D = 128
L = 4
G = 64

NC = 2
NS = 16
NW = NC * NS

EPT = E // NW
K = 80
NCHUNK = EPT // K
NPT = NP // NS


R = 3
RI = 6
P = 4


def _agg_body(h_hbm, src_hbm, dst_hbm, zeros_hbm, out_hbm, *scratch):
    sidx = scratch[0]
    ibuf = scratch[1]
    rows = scratch[2:2 + R]
    acc = scratch[2 + R]
    gsem = scratch[3 + R:3 + 2 * R]
    ssem = scratch[3 + 2 * R:3 + 3 * R]
    isem = scratch[3 + 3 * R:3 + 3 * R + RI]

    c = lax.axis_index("c")
    s = lax.axis_index("s")
    wid = c * NS + s
    ebase = wid * EPT
    nbase = s * NPT

    def _idx_fetch(m, slot):
        pltpu.async_copy(dst_hbm.at[pl.ds(ebase + m * K, K)],
                         ibuf.at[slot], isem[slot])

    pltpu.sync_copy(zeros_hbm.at[pl.ds(nbase, NPT)], acc.at[pl.ds(nbase, NPT)])
    pltpu.sync_copy(src_hbm.at[pl.ds(ebase, EPT)], sidx)
    for m in range(P):
        _idx_fetch(m, m % RI)
    plsc.subcore_barrier()

    def _drain(buf, sem):
        pltpu.make_async_copy(h_hbm.at[pl.ds(0, K)], buf, sem).wait()

    def _draini(q):
        pltpu.make_async_copy(dst_hbm.at[pl.ds(0, K)], ibuf.at[q],
                              isem[q]).wait()

    pltpu.async_copy(h_hbm.at[sidx.at[pl.ds(0, K)]], rows[0], gsem[0])
    pltpu.async_copy(h_hbm.at[sidx.at[pl.ds(K, K)]], rows[1], gsem[1])

    def _stage(j, b, b2, q, fetch, drain_b2, glast):
        if not glast:
            if drain_b2:
                _drain(rows[b2], ssem[b2])
            pltpu.async_copy(
                h_hbm.at[sidx.at[pl.ds((j + 2) * K, K)]], rows[b2], gsem[b2])
        if fetch:
            _idx_fetch(j + P, (q + P) % RI)
        _drain(rows[b], gsem[b])
        _draini(q)
        pltpu.async_copy(rows[b], acc.at[ibuf.at[q]], ssem[b], add=True)

    MAIN = (NCHUNK - 1) // RI * RI

    for r in range(RI):
        _stage(r, r % R, (r + 2) % R, r, True, r >= 1, False)

    @pl.loop(RI, MAIN, step=RI)
    def _chunk(j):
        for r in range(RI):
            _stage(j + r, r % R, (r + 2) % R, r, True, True, False)

    for j in range(MAIN, NCHUNK):
        _stage(j, j % R, (j + 2) % R, j % RI, j + P < NCHUNK, True,
               j + 2 >= NCHUNK)
    for b in range(R):
        _drain(rows[b], ssem[b])

    plsc.subcore_barrier()
    pltpu.sync_copy(acc.at[pl.ds(nbase, NPT)], out_hbm.at[c, pl.ds(nbase, NPT)])


_agg = pl.kernel(
    _agg_body,
    out_type=jax.ShapeDtypeStruct((NC, NP, D), jnp.float32),
    mesh=plsc.VectorSubcoreMesh(
        core_axis_name="c", subcore_axis_name="s",
        num_cores=NC, num_subcores=NS),
    scratch_types=(
        [pltpu.VMEM((EPT,), jnp.int32),
         pltpu.VMEM((RI, K), jnp.int32)]
        + [pltpu.VMEM((K, D), jnp.float32) for _ in range(R)]
        + [pltpu.VMEM_SHARED((NP, D), jnp.float32)]
        + [pltpu.SemaphoreType.DMA for _ in range(2 * R)]
        + [pltpu.SemaphoreType.DMA for _ in range(RI)]
    ),
)


_RB = 1024
_NB = NP // _RB

_row = pl.BlockSpec((_RB, D), lambda i: (i, 0))
_full = pl.BlockSpec((D, D), lambda i: (0, 0))
_bias = pl.BlockSpec((1, D), lambda i: (0, 0))


def _premm_body(h_ref, w1_ref, b1_ref, o_ref):
    o_ref[...] = jnp.dot(h_ref[...], w1_ref[...],
                         preferred_element_type=jnp.float32) + b1_ref[...]


def _premm(h, w1, b1):
    return pl.pallas_call(
        _premm_body,
        grid=(_NB,),
        in_specs=[_row, _full, _bias],
        out_specs=_row,
        out_shape=jax.ShapeDtypeStruct((NP, D), jnp.float32),
    )(h, w1, b1.reshape(1, D))


def _finish(hw1, p0, p1, w1, w2, b2):
    t = hw1 + jnp.dot(p0 + p1, w1, preferred_element_type=jnp.float32)
    t = jnp.maximum(t, 0.0)
    return jnp.dot(t, w2, preferred_element_type=jnp.float32) + b2


def _mlp_body(hw1_ref, p0_ref, p1_ref, w1_ref, w2_ref, b2_ref, o_ref):
    o = _finish(hw1_ref[...], p0_ref[...], p1_ref[...], w1_ref[...],
                w2_ref[...], b2_ref[...])
    o_ref[...] = jnp.maximum(o, 0.0)


def _mlp(hw1, p0, p1, w1, w2, b2):
    return pl.pallas_call(
        _mlp_body,
        grid=(_NB,),
        in_specs=[_row, _row, _row, _full, _full, _bias],
        out_specs=_row,
        out_shape=jax.ShapeDtypeStruct((NP, D), jnp.float32),
    )(hw1, p0, p1, w1, w2, b2.reshape(1, D))


def _mlp_pool_body(hw1_ref, p0_ref, p1_ref, w1_ref, w2_ref, b2_ref, b_ref,
                   o_ref, sums, cnts):
    i = pl.program_id(0)

    @pl.when(i == 0)
    def _():
        sums[...] = jnp.zeros_like(sums)
        cnts[...] = jnp.zeros_like(cnts)

    o = _finish(hw1_ref[...], p0_ref[...], p1_ref[...], w1_ref[...],
                w2_ref[...], b2_ref[...])
    bvec = b_ref[0]
    gids = lax.broadcasted_iota(jnp.int32, (G, _RB), 0)
    m = (bvec == gids).astype(jnp.float32)
    sums[...] += lax.dot_general(m, o, (((1,), (0,)), ((), ())),
                                 preferred_element_type=jnp.float32)
    cnts[...] += lax.dot_general(m, jnp.ones_like(o), (((1,), (0,)), ((), ())),
                                 preferred_element_type=jnp.float32)

    @pl.when(i == _NB - 1)
    def _():
        o_ref[...] = sums[...] / jnp.maximum(cnts[...], 1.0)


def _mlp_pool(hw1, p0, p1, w1, w2, b2, batch3d):
    return pl.pallas_call(
        _mlp_pool_body,
        grid=(_NB,),
        in_specs=[_row, _row, _row, _full, _full, _bias,
                  pl.BlockSpec((1, 1, _RB), lambda i: (i, 0, 0))],
        out_specs=pl.BlockSpec((G, D), lambda i: (0, 0)),
        out_shape=jax.ShapeDtypeStruct((G, D), jnp.float32),
        scratch_shapes=[
            pltpu.VMEM((G, D), jnp.float32),
            pltpu.VMEM((G, D), jnp.float32),
        ],
    )(hw1, p0, p1, w1, w2, b2.reshape(1, D), batch3d)


def kernel(x, edge_index, batch, W1, b1, W2, b2):
    src = edge_index[0]
    dst = edge_index[1]
    zeros = jnp.zeros((NP, D), jnp.float32)
    h = jnp.pad(x, ((0, NP - N), (0, 0)))
    batch3d = jnp.pad(batch, (0, NP - N), constant_values=G).reshape(
        _NB, 1, _RB)
    for l in range(L):
        hw1 = _premm(h, W1[l], b1[l])
        parts = _agg(h, src, dst, zeros)
        if l < L - 1:
            h = _mlp(hw1, parts[0], parts[1], W1[l], W2[l], b2[l])
        else:
            return _mlp_pool(hw1, parts[0], parts[1], W1[l], W2[l], b2[l],
                             batch3d)
